# Initial kernel scaffold; baseline (speedup 1.0000x reference)
import jax
import jax.numpy as jnp
from jax.experimental import pallas as pl


def kernel(x, nodes, nbr1, nbr2, W0l, b0l, W0r, b0r, W1l, b1l, W1r, b1r, Wp, bp):
    raise NotImplementedError("write your pallas kernel here")


# speedup vs baseline: 5.3360x; 5.3360x over previous
import functools

import jax
import jax.numpy as jnp
from jax import lax
from jax.experimental import pallas as pl
from jax.experimental.pallas import tpu as pltpu
from jax.experimental.pallas import tpu_sc as plsc

N_NODES = 10000
D = 128
N0 = 4096
S1 = 5
S2 = 2
H1 = 64
T = 5
NCLS = 64

NC = 2
NS = 16
L = 16
NW = NC * NS
SEEDS_PER_W = N0 // NW
C = 16
NCH = SEEDS_PER_W // C
ROWS1 = S1 * C
ROWS2 = S1 * S2 * C
HALF2 = ROWS2 // 2

_PREC = lax.Precision.HIGHEST


def _prep_body(x_ref, wl_ref, wr_ref, plr_ref, prt_ref):
    xv = x_ref[...]
    dl = lax.dot_general(xv, wl_ref[...], (((1,), (1,)), ((), ())),
                         preferred_element_type=jnp.float32, precision=_PREC)
    dr = lax.dot_general(xv, wr_ref[...], (((1,), (1,)), ((), ())),
                         preferred_element_type=jnp.float32, precision=_PREC)
    plr_ref[:, :D] = dl
    plr_ref[:, D:] = dr
    prt_ref[...] = dr


_RB = 1000

_prep = pl.pallas_call(
    _prep_body,
    grid=(N_NODES // _RB,),
    in_specs=[
        pl.BlockSpec((_RB, D), lambda i: (i, 0)),
        pl.BlockSpec((D, D), lambda i: (0, 0)),
        pl.BlockSpec((D, D), lambda i: (0, 0)),
    ],
    out_specs=[
        pl.BlockSpec((_RB, 2 * D), lambda i: (i, 0)),
        pl.BlockSpec((_RB, D), lambda i: (i, 0)),
    ],
    out_shape=[
        jax.ShapeDtypeStruct((N_NODES, 2 * D), jnp.float32),
        jax.ShapeDtypeStruct((N_NODES, D), jnp.float32),
    ],
)


@functools.partial(
    pl.kernel,
    out_type=jax.ShapeDtypeStruct((T, N0, 2 * D), jnp.float32),
    mesh=plsc.VectorSubcoreMesh(core_axis_name="c", subcore_axis_name="s"),
    scratch_types=[
        pltpu.VMEM((C,), jnp.int32),
        pltpu.VMEM((ROWS1,), jnp.int32),
        pltpu.VMEM((HALF2,), jnp.int32),
        pltpu.VMEM((HALF2,), jnp.int32),
        pltpu.VMEM((C, 2 * D), jnp.float32),
        pltpu.VMEM((ROWS1, 2 * D), jnp.float32),
        pltpu.VMEM((ROWS2, D), jnp.float32),
        pltpu.VMEM((D,), jnp.float32),
        pltpu.VMEM((C, 2 * D), jnp.float32),
        pltpu.SemaphoreType.DMA,
    ],
)
def _sc_gather(plr_hbm, prt_hbm, nodes_hbm, nbr1_hbm, nbr2_hbm, b0_hbm,
               out_hbm, idxn, idx1, idx2a, idx2b, g0, g1, g2, b0v, outb, sem):
    cid = lax.axis_index("c")
    sid = lax.axis_index("s")
    wid = sid * NC + cid
    base = wid * SEEDS_PER_W
    pltpu.sync_copy(b0_hbm, b0v)

    def chunk_body(ch, carry):
        gbase = base + ch * C
        pltpu.sync_copy(nodes_hbm.at[pl.ds(gbase, C)], idxn)
        pltpu.async_copy(plr_hbm.at[idxn], g0, sem).wait()

        def t_body(t, carry_t):
            o1 = t * (N0 * S1) + S1 * gbase
            o2 = t * (N0 * S1 * S2) + S1 * S2 * gbase
            pltpu.sync_copy(nbr1_hbm.at[pl.ds(o1, ROWS1)], idx1)
            pltpu.sync_copy(nbr2_hbm.at[pl.ds(o2, HALF2)], idx2a)
            pltpu.sync_copy(nbr2_hbm.at[pl.ds(o2 + HALF2, HALF2)], idx2b)
            c1 = pltpu.async_copy(plr_hbm.at[idx1], g1, sem)
            c2 = pltpu.async_copy(prt_hbm.at[idx2a], g2.at[pl.ds(0, HALF2)], sem)
            c3 = pltpu.async_copy(prt_hbm.at[idx2b], g2.at[pl.ds(HALF2, HALF2)], sem)
            c1.wait()
            c2.wait()
            c3.wait()

            def s_body(s, carry_s):
                for c in range(D // L):
                    col = c * L
                    b0c = b0v[pl.ds(col, L)]
                    acc = g1[S1 * s, pl.ds(D + col, L)]
                    for j in range(1, S1):
                        acc = acc + g1[S1 * s + j, pl.ds(D + col, L)]
                    so = g0[s, pl.ds(col, L)] + acc * 0.2 + b0c
                    outb[s, pl.ds(col, L)] = jnp.where(so > 1.0, 1.0, 0.0)
                    bs = jnp.zeros((L,), jnp.float32)
                    for j in range(S1):
                        r2 = S1 * S2 * s + S2 * j
                        bv = (g1[S1 * s + j, pl.ds(col, L)]
                              + (g2[r2, pl.ds(col, L)] + g2[r2 + 1, pl.ds(col, L)]) * 0.5
                              + b0c)
                        bs = bs + jnp.where(bv > 1.0, 1.0, 0.0)
                    outb[s, pl.ds(D + col, L)] = bs * 0.2
                return carry_s

            lax.fori_loop(0, C, s_body, 0)
            pltpu.sync_copy(outb, out_hbm.at[t, pl.ds(gbase, C)])
            return carry_t

        lax.fori_loop(0, T, t_body, 0)
        return carry

    lax.fori_loop(0, NCH, chunk_body, 0)


def _head_body(h_ref, w1l_ref, w1r_ref, b1l_ref, b1r_ref, wp_ref, bp_ref,
               out_ref):
    t = pl.program_id(0)
    a = h_ref[0, :, :D]
    bm = h_ref[0, :, D:]
    dl = lax.dot_general(a, w1l_ref[...], (((1,), (1,)), ((), ())),
                         preferred_element_type=jnp.float32, precision=_PREC)
    dr = lax.dot_general(bm, w1r_ref[...], (((1,), (1,)), ((), ())),
                         preferred_element_type=jnp.float32, precision=_PREC)
    out1 = dl + b1l_ref[0] + dr + b1r_ref[0]
    s1 = jnp.where(out1 > 1.0, 1.0, 0.0)
    contrib = lax.dot_general(s1, wp_ref[0], (((1,), (1,)), ((), ())),
                              preferred_element_type=jnp.float32,
                              precision=_PREC)

    @pl.when(t == 0)
    def _():
        out_ref[...] = jnp.broadcast_to(bp_ref[0], (N0, NCLS))

    out_ref[...] += contrib


_head = pl.pallas_call(
    _head_body,
    grid=(T,),
    in_specs=[
        pl.BlockSpec((1, N0, 2 * D), lambda t: (t, 0, 0)),
        pl.BlockSpec((H1, D), lambda t: (0, 0)),
        pl.BlockSpec((H1, D), lambda t: (0, 0)),
        pl.BlockSpec((1, H1), lambda t: (0, 0)),
        pl.BlockSpec((1, H1), lambda t: (0, 0)),
        pl.BlockSpec((1, H1, H1), lambda t: (t, 0, 0)),
        pl.BlockSpec((1, NCLS), lambda t: (0, 0)),
    ],
    out_specs=pl.BlockSpec((N0, NCLS), lambda t: (0, 0)),
    out_shape=jax.ShapeDtypeStruct((N0, NCLS), jnp.float32),
)


def kernel(x, nodes, nbr1, nbr2, W0l, b0l, W0r, b0r, W1l, b1l, W1r, b1r,
           Wp, bp):
    plr, prt = _prep(x, W0l, W0r)
    b0 = b0l + b0r
    h = _sc_gather(plr, prt, nodes, nbr1.reshape(-1), nbr2.reshape(-1), b0)
    wp_t = Wp.reshape(NCLS, T, H1).transpose(1, 0, 2)
    return _head(h, W1l, W1r, b1l.reshape(1, H1), b1r.reshape(1, H1), wp_t,
                 bp.reshape(1, NCLS))


# speedup vs baseline: 8.2785x; 1.5514x over previous
import functools

import jax
import jax.numpy as jnp
from jax import lax
from jax.experimental import pallas as pl
from jax.experimental.pallas import tpu as pltpu
from jax.experimental.pallas import tpu_sc as plsc

N_NODES = 10000
D = 128
N0 = 4096
S1 = 5
S2 = 2
H1 = 64
T = 5
NCLS = 64

NC = 2
NS = 16
L = 16
NW = NC * NS
SEEDS_PER_W = N0 // NW

_PREC = lax.Precision.HIGHEST


def _prep_body(x_ref, wl_ref, wr_ref, plr_ref, prt_ref, plt_ref):
    xv = x_ref[...]
    dl = lax.dot_general(xv, wl_ref[...], (((1,), (1,)), ((), ())),
                         preferred_element_type=jnp.float32, precision=_PREC)
    dr = lax.dot_general(xv, wr_ref[...], (((1,), (1,)), ((), ())),
                         preferred_element_type=jnp.float32, precision=_PREC)
    plr_ref[:, :D] = dl
    plr_ref[:, D:] = dr
    prt_ref[...] = dr
    plt_ref[...] = dl


_RB = 1000

_prep = pl.pallas_call(
    _prep_body,
    grid=(N_NODES // _RB,),
    in_specs=[
        pl.BlockSpec((_RB, D), lambda i: (i, 0)),
        pl.BlockSpec((D, D), lambda i: (0, 0)),
        pl.BlockSpec((D, D), lambda i: (0, 0)),
    ],
    out_specs=[
        pl.BlockSpec((_RB, 2 * D), lambda i: (i, 0)),
        pl.BlockSpec((_RB, D), lambda i: (i, 0)),
        pl.BlockSpec((_RB, D), lambda i: (i, 0)),
    ],
    out_shape=[
        jax.ShapeDtypeStruct((N_NODES, 2 * D), jnp.float32),
        jax.ShapeDtypeStruct((N_NODES, D), jnp.float32),
        jax.ShapeDtypeStruct((N_NODES, D), jnp.float32),
    ],
)


CH = 8
CPT = SEEDS_PER_W // CH
NFLAT = T * CPT
NPAIR = NFLAT // 2
R1C = S1 * CH
R2C = S1 * S2 * CH


@functools.partial(
    pl.kernel,
    out_type=jax.ShapeDtypeStruct((T, N0, 2 * D), jnp.float32),
    mesh=plsc.VectorSubcoreMesh(core_axis_name="c", subcore_axis_name="s"),
    scratch_types=[
        pltpu.VMEM((SEEDS_PER_W,), jnp.int32),
        pltpu.VMEM((T * S1 * SEEDS_PER_W,), jnp.int32),
        pltpu.VMEM((T * S1 * S2 * SEEDS_PER_W,), jnp.int32),
        pltpu.VMEM((SEEDS_PER_W, D), jnp.float32),
        pltpu.VMEM((R1C, 2 * D), jnp.float32),
        pltpu.VMEM((R1C, 2 * D), jnp.float32),
        pltpu.VMEM((R2C, D), jnp.float32),
        pltpu.VMEM((R2C, D), jnp.float32),
        pltpu.VMEM((D,), jnp.float32),
        pltpu.VMEM((SEEDS_PER_W, 2 * D), jnp.float32),
        pltpu.SemaphoreType.DMA,
        pltpu.SemaphoreType.DMA,
        pltpu.SemaphoreType.DMA,
    ],
)
def _sc_gather(plr_hbm, prt_hbm, plt_hbm, nodes_hbm, nbr1_hbm, nbr2_hbm,
               b0_hbm, out_hbm, idxn, idx1, idx2, g0, g1a, g1b, g2a, g2b,
               b0v, outT, semA, semB, sem0):
    cid = lax.axis_index("c")
    sid = lax.axis_index("s")
    wid = sid * NC + cid
    base = wid * SEEDS_PER_W

    n1w = S1 * SEEDS_PER_W
    n2w = S1 * S2 * SEEDS_PER_W
    stage = [
        pltpu.async_copy(b0_hbm, b0v, sem0),
        pltpu.async_copy(nodes_hbm.at[pl.ds(base, SEEDS_PER_W)], idxn, sem0),
    ]
    for t in range(T):
        stage.append(pltpu.async_copy(
            nbr1_hbm.at[pl.ds(t * (N0 * S1) + S1 * base, n1w)],
            idx1.at[pl.ds(t * n1w, n1w)], sem0))
        stage.append(pltpu.async_copy(
            nbr2_hbm.at[pl.ds(t * (N0 * S1 * S2) + S1 * S2 * base, n2w)],
            idx2.at[pl.ds(t * n2w, n2w)], sem0))
    for cp in stage:
        cp.wait()
    pltpu.async_copy(plt_hbm.at[idxn], g0, sem0).wait()

    def issue(i, g1, g2, semX):
        pltpu.async_copy(plr_hbm.at[idx1.at[pl.ds(i * R1C, R1C)]], g1, semX)
        pltpu.async_copy(prt_hbm.at[idx2.at[pl.ds(i * R2C, R2C)]], g2, semX)

    def drain(g1, g2, semX):
        pltpu.make_async_copy(
            plr_hbm.at[idx1.at[pl.ds(0, R1C)]], g1, semX).wait()
        pltpu.make_async_copy(
            prt_hbm.at[idx2.at[pl.ds(0, R2C)]], g2, semX).wait()

    def compute(i, g1, g2):
        row0 = lax.rem(i, CPT) * CH

        def s_body(s, carry_s):
            orow = row0 + s
            for c in range(D // L):
                col = c * L
                b0c = b0v[pl.ds(col, L)]
                acc = g1[S1 * s, pl.ds(D + col, L)]
                for j in range(1, S1):
                    acc = acc + g1[S1 * s + j, pl.ds(D + col, L)]
                so = g0[orow, pl.ds(col, L)] + acc * 0.2 + b0c
                outT[orow, pl.ds(col, L)] = jnp.where(so > 1.0, 1.0, 0.0)
                bs = jnp.zeros((L,), jnp.float32)
                for j in range(S1):
                    r2 = S1 * S2 * s + S2 * j
                    bv = (g1[S1 * s + j, pl.ds(col, L)]
                          + (g2[r2, pl.ds(col, L)]
                             + g2[r2 + 1, pl.ds(col, L)]) * 0.5
                          + b0c)
                    bs = bs + jnp.where(bv > 1.0, 1.0, 0.0)
                outT[orow, pl.ds(D + col, L)] = bs * 0.2
            return carry_s

        lax.fori_loop(0, CH, s_body, 0)

    issue(0, g1a, g2a, semA)

    def g_body(g, carry):
        i0 = 2 * g
        issue(i0 + 1, g1b, g2b, semB)

        @pl.when(jnp.logical_and(lax.rem(g, CPT // 2) == 0, g > 0))
        def _():
            tprev = lax.div(g, CPT // 2) - 1
            pltpu.sync_copy(outT, out_hbm.at[tprev, pl.ds(base, SEEDS_PER_W)])

        drain(g1a, g2a, semA)
        compute(i0, g1a, g2a)

        @pl.when(g < NPAIR - 1)
        def _():
            issue(i0 + 2, g1a, g2a, semA)

        drain(g1b, g2b, semB)
        compute(i0 + 1, g1b, g2b)
        return carry

    lax.fori_loop(0, NPAIR, g_body, 0)
    pltpu.sync_copy(outT, out_hbm.at[T - 1, pl.ds(base, SEEDS_PER_W)])


def _head_body(h_ref, w1l_ref, w1r_ref, b1l_ref, b1r_ref, wp_ref, bp_ref,
               out_ref):
    t = pl.program_id(0)
    a = h_ref[0, :, :D]
    bm = h_ref[0, :, D:]
    dl = lax.dot_general(a, w1l_ref[...], (((1,), (1,)), ((), ())),
                         preferred_element_type=jnp.float32, precision=_PREC)
    dr = lax.dot_general(bm, w1r_ref[...], (((1,), (1,)), ((), ())),
                         preferred_element_type=jnp.float32, precision=_PREC)
    out1 = dl + b1l_ref[0] + dr + b1r_ref[0]
    s1 = jnp.where(out1 > 1.0, 1.0, 0.0)
    contrib = lax.dot_general(s1, wp_ref[0], (((1,), (1,)), ((), ())),
                              preferred_element_type=jnp.float32,
                              precision=_PREC)

    @pl.when(t == 0)
    def _():
        out_ref[...] = jnp.broadcast_to(bp_ref[0], (N0, NCLS))

    out_ref[...] += contrib


_head = pl.pallas_call(
    _head_body,
    grid=(T,),
    in_specs=[
        pl.BlockSpec((1, N0, 2 * D), lambda t: (t, 0, 0)),
        pl.BlockSpec((H1, D), lambda t: (0, 0)),
        pl.BlockSpec((H1, D), lambda t: (0, 0)),
        pl.BlockSpec((1, H1), lambda t: (0, 0)),
        pl.BlockSpec((1, H1), lambda t: (0, 0)),
        pl.BlockSpec((1, H1, H1), lambda t: (t, 0, 0)),
        pl.BlockSpec((1, NCLS), lambda t: (0, 0)),
    ],
    out_specs=pl.BlockSpec((N0, NCLS), lambda t: (0, 0)),
    out_shape=jax.ShapeDtypeStruct((N0, NCLS), jnp.float32),
)


def kernel(x, nodes, nbr1, nbr2, W0l, b0l, W0r, b0r, W1l, b1l, W1r, b1r,
           Wp, bp):
    plr, prt, plt = _prep(x, W0l, W0r)
    b0 = b0l + b0r
    h = _sc_gather(plr, prt, plt, nodes, nbr1.reshape(-1), nbr2.reshape(-1),
                   b0)
    wp_t = Wp.reshape(NCLS, T, H1).transpose(1, 0, 2)
    return _head(h, W1l, W1r, b1l.reshape(1, H1), b1r.reshape(1, H1), wp_t,
                 bp.reshape(1, NCLS))


# speedup vs baseline: 9.2031x; 1.1117x over previous
import functools

import jax
import jax.numpy as jnp
from jax import lax
from jax.experimental import pallas as pl
from jax.experimental.pallas import tpu as pltpu
from jax.experimental.pallas import tpu_sc as plsc

N_NODES = 10000
D = 128
N0 = 4096
S1 = 5
S2 = 2
H1 = 64
T = 5
NCLS = 64

NC = 2
NS = 16
L = 16
NW = NC * NS
SEEDS_PER_W = N0 // NW

_PREC = lax.Precision.HIGHEST


def _prep_body(x_ref, wl_ref, wr_ref, b0_ref, plr_ref, prt_ref, plt_ref):
    xv = x_ref[...]
    dl = lax.dot_general(xv, wl_ref[...], (((1,), (1,)), ((), ())),
                         preferred_element_type=jnp.float32, precision=_PREC)
    dr = lax.dot_general(xv, wr_ref[...], (((1,), (1,)), ((), ())),
                         preferred_element_type=jnp.float32, precision=_PREC)
    plb = dl + b0_ref[0]
    plr_ref[:, :D] = plb
    plr_ref[:, D:] = dr * 0.2
    prt_ref[...] = dr * 0.5
    plt_ref[...] = plb


_RB = 1000

_prep = pl.pallas_call(
    _prep_body,
    grid=(N_NODES // _RB,),
    in_specs=[
        pl.BlockSpec((_RB, D), lambda i: (i, 0)),
        pl.BlockSpec((D, D), lambda i: (0, 0)),
        pl.BlockSpec((D, D), lambda i: (0, 0)),
        pl.BlockSpec((1, D), lambda i: (0, 0)),
    ],
    out_specs=[
        pl.BlockSpec((_RB, 2 * D), lambda i: (i, 0)),
        pl.BlockSpec((_RB, D), lambda i: (i, 0)),
        pl.BlockSpec((_RB, D), lambda i: (i, 0)),
    ],
    out_shape=[
        jax.ShapeDtypeStruct((N_NODES, 2 * D), jnp.float32),
        jax.ShapeDtypeStruct((N_NODES, D), jnp.float32),
        jax.ShapeDtypeStruct((N_NODES, D), jnp.float32),
    ],
)


CH = 8
CPT = SEEDS_PER_W // CH
NFLAT = T * CPT
NPAIR = NFLAT // 2
R1C = S1 * CH
R2C = S1 * S2 * CH


@functools.partial(
    pl.kernel,
    out_type=jax.ShapeDtypeStruct((T, N0, 2 * D), jnp.float32),
    mesh=plsc.VectorSubcoreMesh(core_axis_name="c", subcore_axis_name="s"),
    scratch_types=[
        pltpu.VMEM((SEEDS_PER_W,), jnp.int32),
        pltpu.VMEM((T * S1 * SEEDS_PER_W,), jnp.int32),
        pltpu.VMEM((T * S1 * S2 * SEEDS_PER_W,), jnp.int32),
        pltpu.VMEM((SEEDS_PER_W, D), jnp.float32),
        pltpu.VMEM((R1C, 2 * D), jnp.float32),
        pltpu.VMEM((R1C, 2 * D), jnp.float32),
        pltpu.VMEM((R2C, D), jnp.float32),
        pltpu.VMEM((R2C, D), jnp.float32),
        pltpu.VMEM((SEEDS_PER_W, 2 * D), jnp.float32),
        pltpu.SemaphoreType.DMA,
        pltpu.SemaphoreType.DMA,
        pltpu.SemaphoreType.DMA,
    ],
)
def _sc_gather(plr_hbm, prt_hbm, plt_hbm, nodes_hbm, nbr1_hbm, nbr2_hbm,
               out_hbm, idxn, idx1, idx2, g0, g1a, g1b, g2a, g2b,
               outT, semA, semB, sem0):
    cid = lax.axis_index("c")
    sid = lax.axis_index("s")
    wid = sid * NC + cid
    base = wid * SEEDS_PER_W

    n1w = S1 * SEEDS_PER_W
    n2w = S1 * S2 * SEEDS_PER_W
    stage = [
        pltpu.async_copy(nodes_hbm.at[pl.ds(base, SEEDS_PER_W)], idxn, sem0),
    ]
    for t in range(T):
        stage.append(pltpu.async_copy(
            nbr1_hbm.at[pl.ds(t * (N0 * S1) + S1 * base, n1w)],
            idx1.at[pl.ds(t * n1w, n1w)], sem0))
        stage.append(pltpu.async_copy(
            nbr2_hbm.at[pl.ds(t * (N0 * S1 * S2) + S1 * S2 * base, n2w)],
            idx2.at[pl.ds(t * n2w, n2w)], sem0))
    for cp in stage:
        cp.wait()
    pltpu.async_copy(plt_hbm.at[idxn], g0, sem0).wait()

    def issue(i, g1, g2, semX):
        pltpu.async_copy(plr_hbm.at[idx1.at[pl.ds(i * R1C, R1C)]], g1, semX)
        pltpu.async_copy(prt_hbm.at[idx2.at[pl.ds(i * R2C, R2C)]], g2, semX)

    def drain(g1, g2, semX):
        pltpu.make_async_copy(
            plr_hbm.at[idx1.at[pl.ds(0, R1C)]], g1, semX).wait()
        pltpu.make_async_copy(
            prt_hbm.at[idx2.at[pl.ds(0, R2C)]], g2, semX).wait()

    def compute(i, g1, g2):
        row0 = lax.rem(i, CPT) * CH

        def s_body(s, carry_s):
            orow = row0 + s
            for c in range(D // L):
                col = c * L
                acc = g1[S1 * s, pl.ds(D + col, L)]
                for j in range(1, S1):
                    acc = acc + g1[S1 * s + j, pl.ds(D + col, L)]
                so = g0[orow, pl.ds(col, L)] + acc
                outT[orow, pl.ds(col, L)] = jnp.where(so > 1.0, 1.0, 0.0)
                bs = jnp.zeros((L,), jnp.float32)
                for j in range(S1):
                    r2 = S1 * S2 * s + S2 * j
                    bv = (g1[S1 * s + j, pl.ds(col, L)]
                          + g2[r2, pl.ds(col, L)]
                          + g2[r2 + 1, pl.ds(col, L)])
                    bs = bs + jnp.where(bv > 1.0, 1.0, 0.0)
                outT[orow, pl.ds(D + col, L)] = bs
            return carry_s

        lax.fori_loop(0, CH, s_body, 0)

    issue(0, g1a, g2a, semA)

    def g_body(g, carry):
        i0 = 2 * g
        issue(i0 + 1, g1b, g2b, semB)

        @pl.when(jnp.logical_and(lax.rem(g, CPT // 2) == 0, g > 0))
        def _():
            tprev = lax.div(g, CPT // 2) - 1
            pltpu.sync_copy(outT, out_hbm.at[tprev, pl.ds(base, SEEDS_PER_W)])

        drain(g1a, g2a, semA)
        compute(i0, g1a, g2a)

        @pl.when(g < NPAIR - 1)
        def _():
            issue(i0 + 2, g1a, g2a, semA)

        drain(g1b, g2b, semB)
        compute(i0 + 1, g1b, g2b)
        return carry

    lax.fori_loop(0, NPAIR, g_body, 0)
    pltpu.sync_copy(outT, out_hbm.at[T - 1, pl.ds(base, SEEDS_PER_W)])


def _head_body(h_ref, w1l_ref, w1r_ref, b1l_ref, b1r_ref, wp_ref, bp_ref,
               out_ref):
    t = pl.program_id(0)
    a = h_ref[0, :, :D]
    bm = h_ref[0, :, D:]
    dl = lax.dot_general(a, w1l_ref[...], (((1,), (1,)), ((), ())),
                         preferred_element_type=jnp.float32, precision=_PREC)
    dr = lax.dot_general(bm, w1r_ref[...] * 0.2, (((1,), (1,)), ((), ())),
                         preferred_element_type=jnp.float32, precision=_PREC)
    out1 = dl + b1l_ref[0] + dr + b1r_ref[0]
    s1 = jnp.where(out1 > 1.0, 1.0, 0.0)
    contrib = lax.dot_general(s1, wp_ref[0], (((1,), (1,)), ((), ())),
                              preferred_element_type=jnp.float32,
                              precision=_PREC)

    @pl.when(t == 0)
    def _():
        out_ref[...] = jnp.broadcast_to(bp_ref[0], (N0, NCLS))

    out_ref[...] += contrib


_head = pl.pallas_call(
    _head_body,
    grid=(T,),
    in_specs=[
        pl.BlockSpec((1, N0, 2 * D), lambda t: (t, 0, 0)),
        pl.BlockSpec((H1, D), lambda t: (0, 0)),
        pl.BlockSpec((H1, D), lambda t: (0, 0)),
        pl.BlockSpec((1, H1), lambda t: (0, 0)),
        pl.BlockSpec((1, H1), lambda t: (0, 0)),
        pl.BlockSpec((1, H1, H1), lambda t: (t, 0, 0)),
        pl.BlockSpec((1, NCLS), lambda t: (0, 0)),
    ],
    out_specs=pl.BlockSpec((N0, NCLS), lambda t: (0, 0)),
    out_shape=jax.ShapeDtypeStruct((N0, NCLS), jnp.float32),
)


def kernel(x, nodes, nbr1, nbr2, W0l, b0l, W0r, b0r, W1l, b1l, W1r, b1r,
           Wp, bp):
    b0 = (b0l + b0r).reshape(1, D)
    plr, prt, plt = _prep(x, W0l, W0r, b0)
    h = _sc_gather(plr, prt, plt, nodes, nbr1.reshape(-1), nbr2.reshape(-1))
    wp_t = Wp.reshape(NCLS, T, H1).transpose(1, 0, 2)
    return _head(h, W1l, W1r, b1l.reshape(1, H1), b1r.reshape(1, H1), wp_t,
                 bp.reshape(1, NCLS))


# speedup vs baseline: 10.2813x; 1.1172x over previous
import functools

import jax
import jax.numpy as jnp
from jax import lax
from jax.experimental import pallas as pl
from jax.experimental.pallas import tpu as pltpu
from jax.experimental.pallas import tpu_sc as plsc

N_NODES = 10000
D = 128
N0 = 4096
S1 = 5
S2 = 2
H1 = 64
T = 5
NCLS = 64

NC = 2
NS = 16
L = 16
NW = NC * NS
SEEDS_PER_W = N0 // NW

_PREC = lax.Precision.HIGHEST


def _prep_body(x_ref, wl_ref, wr_ref, b0_ref, plr_ref, prt_ref, plt_ref):
    xv = x_ref[...]
    dl = lax.dot_general(xv, wl_ref[...], (((1,), (1,)), ((), ())),
                         preferred_element_type=jnp.float32, precision=_PREC)
    dr = lax.dot_general(xv, wr_ref[...], (((1,), (1,)), ((), ())),
                         preferred_element_type=jnp.float32, precision=_PREC)
    plb = dl + b0_ref[0]
    plr_ref[:, :D] = plb
    plr_ref[:, D:] = dr * 0.2
    prt_ref[...] = dr * 0.5
    plt_ref[...] = plb


_RB = 5000

_prep = pl.pallas_call(
    _prep_body,
    grid=(N_NODES // _RB,),
    in_specs=[
        pl.BlockSpec((_RB, D), lambda i: (i, 0)),
        pl.BlockSpec((D, D), lambda i: (0, 0)),
        pl.BlockSpec((D, D), lambda i: (0, 0)),
        pl.BlockSpec((1, D), lambda i: (0, 0)),
    ],
    out_specs=[
        pl.BlockSpec((_RB, 2 * D), lambda i: (i, 0)),
        pl.BlockSpec((_RB, D), lambda i: (i, 0)),
        pl.BlockSpec((_RB, D), lambda i: (i, 0)),
    ],
    out_shape=[
        jax.ShapeDtypeStruct((N_NODES, 2 * D), jnp.float32),
        jax.ShapeDtypeStruct((N_NODES, D), jnp.float32),
        jax.ShapeDtypeStruct((N_NODES, D), jnp.float32),
    ],
)


CH = 8
CPT = SEEDS_PER_W // CH
NFLAT = T * CPT
NPAIR = NFLAT // 2
R1C = S1 * CH
R2C = S1 * S2 * CH


@functools.partial(
    pl.kernel,
    out_type=jax.ShapeDtypeStruct((T, N0, 2 * D), jnp.float32),
    mesh=plsc.VectorSubcoreMesh(core_axis_name="c", subcore_axis_name="s"),
    scratch_types=[
        pltpu.VMEM((SEEDS_PER_W,), jnp.int32),
        pltpu.VMEM((T * S1 * SEEDS_PER_W,), jnp.int32),
        pltpu.VMEM((T * S1 * S2 * SEEDS_PER_W,), jnp.int32),
        pltpu.VMEM((SEEDS_PER_W, D), jnp.float32),
        pltpu.VMEM((R1C, 2 * D), jnp.float32),
        pltpu.VMEM((R1C, 2 * D), jnp.float32),
        pltpu.VMEM((R2C, D), jnp.float32),
        pltpu.VMEM((R2C, D), jnp.float32),
        pltpu.VMEM((SEEDS_PER_W, 2 * D), jnp.float32),
        pltpu.SemaphoreType.DMA,
        pltpu.SemaphoreType.DMA,
        pltpu.SemaphoreType.DMA,
    ],
)
def _sc_gather(plr_hbm, prt_hbm, plt_hbm, nodes_hbm, nbr1_hbm, nbr2_hbm,
               out_hbm, idxn, idx1, idx2, g0, g1a, g1b, g2a, g2b,
               outT, semA, semB, sem0):
    cid = lax.axis_index("c")
    sid = lax.axis_index("s")
    wid = sid * NC + cid
    base = wid * SEEDS_PER_W

    n1w = S1 * SEEDS_PER_W
    n2w = S1 * S2 * SEEDS_PER_W
    stage = [
        pltpu.async_copy(nodes_hbm.at[pl.ds(base, SEEDS_PER_W)], idxn, sem0),
    ]
    for t in range(T):
        stage.append(pltpu.async_copy(
            nbr1_hbm.at[pl.ds(t * (N0 * S1) + S1 * base, n1w)],
            idx1.at[pl.ds(t * n1w, n1w)], sem0))
        stage.append(pltpu.async_copy(
            nbr2_hbm.at[pl.ds(t * (N0 * S1 * S2) + S1 * S2 * base, n2w)],
            idx2.at[pl.ds(t * n2w, n2w)], sem0))
    for cp in stage:
        cp.wait()
    pltpu.async_copy(plt_hbm.at[idxn], g0, sem0).wait()

    def issue(i, g1, g2, semX):
        pltpu.async_copy(plr_hbm.at[idx1.at[pl.ds(i * R1C, R1C)]], g1, semX)
        pltpu.async_copy(prt_hbm.at[idx2.at[pl.ds(i * R2C, R2C)]], g2, semX)

    def drain(g1, g2, semX):
        pltpu.make_async_copy(
            plr_hbm.at[idx1.at[pl.ds(0, R1C)]], g1, semX).wait()
        pltpu.make_async_copy(
            prt_hbm.at[idx2.at[pl.ds(0, R2C)]], g2, semX).wait()

    def compute(i, g1, g2):
        row0 = lax.rem(i, CPT) * CH

        def s_body(s, carry_s):
            orow = row0 + s
            for c in range(D // L):
                col = c * L
                a = [g1[S1 * s + j, pl.ds(D + col, L)] for j in range(S1)]
                b = [g1[S1 * s + j, pl.ds(col, L)] for j in range(S1)]
                p = [g2[S1 * S2 * s + k, pl.ds(col, L)]
                     for k in range(S1 * S2)]
                g0v = g0[orow, pl.ds(col, L)]
                so = ((a[0] + a[1]) + (a[2] + a[3])) + (a[4] + g0v)
                outT[orow, pl.ds(col, L)] = jnp.where(so > 1.0, 1.0, 0.0)
                sp = [jnp.where(b[j] + (p[2 * j] + p[2 * j + 1]) > 1.0,
                                1.0, 0.0) for j in range(S1)]
                outT[orow, pl.ds(D + col, L)] = (
                    (sp[0] + sp[1]) + (sp[2] + sp[3])) + sp[4]
            return carry_s

        lax.fori_loop(0, CH, s_body, 0)

    issue(0, g1a, g2a, semA)

    def g_body(g, carry):
        i0 = 2 * g
        issue(i0 + 1, g1b, g2b, semB)

        @pl.when(jnp.logical_and(lax.rem(g, CPT // 2) == 0, g > 0))
        def _():
            tprev = lax.div(g, CPT // 2) - 1
            pltpu.sync_copy(outT, out_hbm.at[tprev, pl.ds(base, SEEDS_PER_W)])

        drain(g1a, g2a, semA)
        compute(i0, g1a, g2a)

        @pl.when(g < NPAIR - 1)
        def _():
            issue(i0 + 2, g1a, g2a, semA)

        drain(g1b, g2b, semB)
        compute(i0 + 1, g1b, g2b)
        return carry

    lax.fori_loop(0, NPAIR, g_body, 0)
    pltpu.sync_copy(outT, out_hbm.at[T - 1, pl.ds(base, SEEDS_PER_W)])


def _head_body(h_ref, w1l_ref, w1r_ref, b1l_ref, b1r_ref, wp_ref, bp_ref,
               out_ref):
    t = pl.program_id(0)
    a = h_ref[0, :, :D]
    bm = h_ref[0, :, D:]
    dl = lax.dot_general(a, w1l_ref[...], (((1,), (1,)), ((), ())),
                         preferred_element_type=jnp.float32, precision=_PREC)
    dr = lax.dot_general(bm, w1r_ref[...] * 0.2, (((1,), (1,)), ((), ())),
                         preferred_element_type=jnp.float32, precision=_PREC)
    out1 = dl + b1l_ref[0] + dr + b1r_ref[0]
    s1 = jnp.where(out1 > 1.0, 1.0, 0.0)
    contrib = lax.dot_general(s1, wp_ref[0], (((1,), (1,)), ((), ())),
                              preferred_element_type=jnp.float32,
                              precision=_PREC)

    @pl.when(t == 0)
    def _():
        out_ref[...] = jnp.broadcast_to(bp_ref[0], (N0, NCLS))

    out_ref[...] += contrib


_head = pl.pallas_call(
    _head_body,
    grid=(T,),
    in_specs=[
        pl.BlockSpec((1, N0, 2 * D), lambda t: (t, 0, 0)),
        pl.BlockSpec((H1, D), lambda t: (0, 0)),
        pl.BlockSpec((H1, D), lambda t: (0, 0)),
        pl.BlockSpec((1, H1), lambda t: (0, 0)),
        pl.BlockSpec((1, H1), lambda t: (0, 0)),
        pl.BlockSpec((1, H1, H1), lambda t: (t, 0, 0)),
        pl.BlockSpec((1, NCLS), lambda t: (0, 0)),
    ],
    out_specs=pl.BlockSpec((N0, NCLS), lambda t: (0, 0)),
    out_shape=jax.ShapeDtypeStruct((N0, NCLS), jnp.float32),
)


def kernel(x, nodes, nbr1, nbr2, W0l, b0l, W0r, b0r, W1l, b1l, W1r, b1r,
           Wp, bp):
    b0 = (b0l + b0r).reshape(1, D)
    plr, prt, plt = _prep(x, W0l, W0r, b0)
    h = _sc_gather(plr, prt, plt, nodes, nbr1.reshape(-1), nbr2.reshape(-1))
    wp_t = Wp.reshape(NCLS, T, H1).transpose(1, 0, 2)
    return _head(h, W1l, W1r, b1l.reshape(1, H1), b1r.reshape(1, H1), wp_t,
                 bp.reshape(1, NCLS))


# speedup vs baseline: 11.0417x; 1.0740x over previous
import functools

import jax
import jax.numpy as jnp
from jax import lax
from jax.experimental import pallas as pl
from jax.experimental.pallas import tpu as pltpu
from jax.experimental.pallas import tpu_sc as plsc

N_NODES = 10000
D = 128
N0 = 4096
S1 = 5
S2 = 2
H1 = 64
T = 5
NCLS = 64

NC = 2
NS = 16
L = 16
NW = NC * NS
SEEDS_PER_W = N0 // NW

_PREC = lax.Precision.HIGHEST


def _prep_body(x_ref, wl_ref, wr_ref, b0_ref, plr_ref, prt_ref, plt_ref):
    xv = x_ref[...]
    dl = lax.dot_general(xv, wl_ref[...], (((1,), (1,)), ((), ())),
                         preferred_element_type=jnp.float32, precision=_PREC)
    dr = lax.dot_general(xv, wr_ref[...], (((1,), (1,)), ((), ())),
                         preferred_element_type=jnp.float32, precision=_PREC)
    plb = dl + b0_ref[0]
    plr_ref[:, :D] = plb
    plr_ref[:, D:] = dr * 0.2
    prt_ref[...] = dr * 0.5
    plt_ref[...] = plb


_RB = 5000

_prep = pl.pallas_call(
    _prep_body,
    grid=(N_NODES // _RB,),
    in_specs=[
        pl.BlockSpec((_RB, D), lambda i: (i, 0)),
        pl.BlockSpec((D, D), lambda i: (0, 0)),
        pl.BlockSpec((D, D), lambda i: (0, 0)),
        pl.BlockSpec((1, D), lambda i: (0, 0)),
    ],
    out_specs=[
        pl.BlockSpec((_RB, 2 * D), lambda i: (i, 0)),
        pl.BlockSpec((_RB, D), lambda i: (i, 0)),
        pl.BlockSpec((_RB, D), lambda i: (i, 0)),
    ],
    out_shape=[
        jax.ShapeDtypeStruct((N_NODES, 2 * D), jnp.float32),
        jax.ShapeDtypeStruct((N_NODES, D), jnp.float32),
        jax.ShapeDtypeStruct((N_NODES, D), jnp.float32),
    ],
)


CH = 8
CPT = SEEDS_PER_W // CH
NFLAT = T * CPT
NPAIR = NFLAT // 2
R1C = S1 * CH
R2C = S1 * S2 * CH


@functools.partial(
    pl.kernel,
    out_type=jax.ShapeDtypeStruct((T, N0, 2 * D), jnp.float32),
    mesh=plsc.VectorSubcoreMesh(core_axis_name="c", subcore_axis_name="s"),
    scratch_types=[
        pltpu.VMEM((SEEDS_PER_W,), jnp.int32),
        pltpu.VMEM((T * S1 * SEEDS_PER_W,), jnp.int32),
        pltpu.VMEM((T * S1 * S2 * SEEDS_PER_W,), jnp.int32),
        pltpu.VMEM((SEEDS_PER_W, D), jnp.float32),
        pltpu.VMEM((R1C, 2 * D), jnp.float32),
        pltpu.VMEM((R1C, 2 * D), jnp.float32),
        pltpu.VMEM((R2C, D), jnp.float32),
        pltpu.VMEM((R2C, D), jnp.float32),
        pltpu.VMEM((SEEDS_PER_W, 2 * D), jnp.float32),
        pltpu.SemaphoreType.DMA,
        pltpu.SemaphoreType.DMA,
        pltpu.SemaphoreType.DMA,
    ],
)
def _sc_gather(plr_hbm, prt_hbm, plt_hbm, nodes_hbm, nbr1_hbm, nbr2_hbm,
               out_hbm, idxn, idx1, idx2, g0, g1a, g1b, g2a, g2b,
               outT, semA, semB, sem0):
    cid = lax.axis_index("c")
    sid = lax.axis_index("s")
    wid = sid * NC + cid
    base = wid * SEEDS_PER_W

    n1w = S1 * SEEDS_PER_W
    n2w = S1 * S2 * SEEDS_PER_W
    stage = [
        pltpu.async_copy(nodes_hbm.at[pl.ds(base, SEEDS_PER_W)], idxn, sem0),
    ]
    for t in range(T):
        stage.append(pltpu.async_copy(
            nbr1_hbm.at[pl.ds(t * (N0 * S1) + S1 * base, n1w)],
            idx1.at[pl.ds(t * n1w, n1w)], sem0))
        stage.append(pltpu.async_copy(
            nbr2_hbm.at[pl.ds(t * (N0 * S1 * S2) + S1 * S2 * base, n2w)],
            idx2.at[pl.ds(t * n2w, n2w)], sem0))
    for cp in stage:
        cp.wait()
    pltpu.async_copy(plt_hbm.at[idxn], g0, sem0).wait()

    def issue(i, g1, g2, semX):
        pltpu.async_copy(plr_hbm.at[idx1.at[pl.ds(i * R1C, R1C)]], g1, semX)
        pltpu.async_copy(prt_hbm.at[idx2.at[pl.ds(i * R2C, R2C)]], g2, semX)

    def drain(g1, g2, semX):
        pltpu.make_async_copy(
            plr_hbm.at[idx1.at[pl.ds(0, R1C)]], g1, semX).wait()
        pltpu.make_async_copy(
            prt_hbm.at[idx2.at[pl.ds(0, R2C)]], g2, semX).wait()

    def compute(i, g1, g2):
        row0 = lax.rem(i, CPT) * CH

        NB = D // L

        def ld_a(s, c):
            col = c * L
            return ([g1[S1 * s + j, pl.ds(D + col, L)] for j in range(S1)],
                    g0[row0 + s, pl.ds(col, L)])

        def ld_nj(s, c, j):
            col = c * L
            return (g1[S1 * s + j, pl.ds(col, L)],
                    g2[S1 * S2 * s + S2 * j, pl.ds(col, L)],
                    g2[S1 * S2 * s + S2 * j + 1, pl.ds(col, L)])

        def s_body(s, carry_s):
            orow = row0 + s
            a_c, g0_c = ld_a(s, 0)
            n_c = [ld_nj(s, 0, j) for j in range(S1)]
            for c in range(NB):
                col = c * L
                nxt = c + 1 < NB
                if nxt:
                    a_n, g0_n = ld_a(s, c + 1)
                so = ((a_c[0] + a_c[1]) + (a_c[2] + a_c[3])) + (a_c[4] + g0_c)
                outT[orow, pl.ds(col, L)] = jnp.where(so > 1.0, 1.0, 0.0)
                sp = []
                n_n = []
                for j in range(S1):
                    if nxt:
                        n_n.append(ld_nj(s, c + 1, j))
                    bj, pj0, pj1 = n_c[j]
                    sp.append(jnp.where(bj + (pj0 + pj1) > 1.0, 1.0, 0.0))
                outT[orow, pl.ds(D + col, L)] = (
                    (sp[0] + sp[1]) + (sp[2] + sp[3])) + sp[4]
                if nxt:
                    a_c, g0_c, n_c = a_n, g0_n, n_n
            return carry_s

        lax.fori_loop(0, CH, s_body, 0)

    issue(0, g1a, g2a, semA)

    def g_body(g, carry):
        i0 = 2 * g
        issue(i0 + 1, g1b, g2b, semB)

        @pl.when(jnp.logical_and(lax.rem(g, CPT // 2) == 0, g > 0))
        def _():
            tprev = lax.div(g, CPT // 2) - 1
            pltpu.sync_copy(outT, out_hbm.at[tprev, pl.ds(base, SEEDS_PER_W)])

        drain(g1a, g2a, semA)
        compute(i0, g1a, g2a)

        @pl.when(g < NPAIR - 1)
        def _():
            issue(i0 + 2, g1a, g2a, semA)

        drain(g1b, g2b, semB)
        compute(i0 + 1, g1b, g2b)
        return carry

    lax.fori_loop(0, NPAIR, g_body, 0)
    pltpu.sync_copy(outT, out_hbm.at[T - 1, pl.ds(base, SEEDS_PER_W)])


def _head_body(h_ref, w1l_ref, w1r_ref, b1l_ref, b1r_ref, wp_ref, bp_ref,
               out_ref):
    t = pl.program_id(0)
    a = h_ref[0, :, :D]
    bm = h_ref[0, :, D:]
    dl = lax.dot_general(a, w1l_ref[...], (((1,), (1,)), ((), ())),
                         preferred_element_type=jnp.float32, precision=_PREC)
    dr = lax.dot_general(bm, w1r_ref[...] * 0.2, (((1,), (1,)), ((), ())),
                         preferred_element_type=jnp.float32, precision=_PREC)
    out1 = dl + b1l_ref[0] + dr + b1r_ref[0]
    s1 = jnp.where(out1 > 1.0, 1.0, 0.0)
    contrib = lax.dot_general(s1, wp_ref[0], (((1,), (1,)), ((), ())),
                              preferred_element_type=jnp.float32,
                              precision=_PREC)

    @pl.when(t == 0)
    def _():
        out_ref[...] = jnp.broadcast_to(bp_ref[0], (N0, NCLS))

    out_ref[...] += contrib


_head = pl.pallas_call(
    _head_body,
    grid=(T,),
    in_specs=[
        pl.BlockSpec((1, N0, 2 * D), lambda t: (t, 0, 0)),
        pl.BlockSpec((H1, D), lambda t: (0, 0)),
        pl.BlockSpec((H1, D), lambda t: (0, 0)),
        pl.BlockSpec((1, H1), lambda t: (0, 0)),
        pl.BlockSpec((1, H1), lambda t: (0, 0)),
        pl.BlockSpec((1, H1, H1), lambda t: (t, 0, 0)),
        pl.BlockSpec((1, NCLS), lambda t: (0, 0)),
    ],
    out_specs=pl.BlockSpec((N0, NCLS), lambda t: (0, 0)),
    out_shape=jax.ShapeDtypeStruct((N0, NCLS), jnp.float32),
)


def kernel(x, nodes, nbr1, nbr2, W0l, b0l, W0r, b0r, W1l, b1l, W1r, b1r,
           Wp, bp):
    b0 = (b0l + b0r).reshape(1, D)
    plr, prt, plt = _prep(x, W0l, W0r, b0)
    h = _sc_gather(plr, prt, plt, nodes, nbr1.reshape(-1), nbr2.reshape(-1))
    wp_t = Wp.reshape(NCLS, T, H1).transpose(1, 0, 2)
    return _head(h, W1l, W1r, b1l.reshape(1, H1), b1r.reshape(1, H1), wp_t,
                 bp.reshape(1, NCLS))
